# baseline (speedup 1.0000x reference)
import functools

import jax
import jax.numpy as jnp
from jax import lax
from jax.experimental import pallas as pl
from jax.experimental.pallas import tpu as pltpu
from jax.experimental.pallas import tpu_sc as plsc

E_DIM = 64
BATCH = 16384
NUM_CORES = 2
NUM_SUBCORES = 16
NUM_WORKERS = NUM_CORES * NUM_SUBCORES
B_PER_W = BATCH // NUM_WORKERS
CHUNK = 128
N_CHUNKS = B_PER_W // CHUNK
LANES = 16


def _vsqrt(x):
    xi = lax.bitcast_convert_type(x, jnp.int32)
    yi = lax.shift_right_logical(xi, 1) + jnp.int32(0x1FBD1DF5)
    y = lax.bitcast_convert_type(yi, jnp.float32)
    for _ in range(3):
        y = 0.5 * (y + x / y)
    return y


def _dist_chunk(h_ref, l_ref, t_ref, out_ref):
    iota = lax.iota(jnp.int32, LANES)

    def group_body(g, _):
        def row_body(i, acc):
            r = g * LANES + i
            p = jnp.zeros((LANES,), jnp.float32)
            for k in range(E_DIM // LANES):
                sl = pl.ds(k * LANES, LANES)
                e = h_ref[r, sl] + l_ref[r, sl] - t_ref[r, sl]
                p = p + e * e
            s = jnp.sum(p)
            return jnp.where(iota == i, s, acc)

        acc = lax.fori_loop(0, LANES, row_body,
                            jnp.zeros((LANES,), jnp.float32))
        out_ref[pl.ds(g * LANES, LANES)] = _vsqrt(acc)
        return 0

    lax.fori_loop(0, CHUNK // LANES, group_body, 0)


def _make_kernel():
    f32 = jnp.float32
    mesh = plsc.VectorSubcoreMesh(core_axis_name="c", subcore_axis_name="s")
    out_type = (
        jax.ShapeDtypeStruct((BATCH,), f32),
        jax.ShapeDtypeStruct((BATCH,), f32),
        jax.ShapeDtypeStruct((BATCH, E_DIM), f32),
        jax.ShapeDtypeStruct((BATCH, E_DIM), f32),
        jax.ShapeDtypeStruct((BATCH, E_DIM), f32),
        jax.ShapeDtypeStruct((BATCH, E_DIM), f32),
    )
    scratch = [
        pltpu.VMEM((CHUNK,), jnp.int32),
        pltpu.VMEM((CHUNK,), jnp.int32),
        pltpu.VMEM((CHUNK,), jnp.int32),
        pltpu.VMEM((CHUNK,), jnp.int32),
        pltpu.VMEM((CHUNK,), jnp.int32),
        pltpu.VMEM((CHUNK,), jnp.int32),
        pltpu.VMEM((CHUNK, E_DIM), f32),
        pltpu.VMEM((CHUNK, E_DIM), f32),
        pltpu.VMEM((CHUNK, E_DIM), f32),
        pltpu.VMEM((CHUNK, E_DIM), f32),
        pltpu.VMEM((CHUNK, E_DIM), f32),
        pltpu.VMEM((CHUNK, E_DIM), f32),
        pltpu.VMEM((CHUNK,), f32),
        pltpu.VMEM((CHUNK,), f32),
        pltpu.SemaphoreType.DMA,
    ]

    @functools.partial(pl.kernel, mesh=mesh, out_type=out_type,
                       scratch_types=scratch,
                       compiler_params=pltpu.CompilerParams(
                           needs_layout_passes=False,
                           use_tc_tiling_on_sc=False))
    def trans_e(h_i, t_i, l_i, ha_i, ta_i, la_i, ent, rel,
                dist_o, dista_o, hv_o, tv_o, hav_o, tav_o,
                ih, it, il, iha, ita, ila,
                rh, rt, rl, rha, rta, rla,
                db, dab, sem):
        wid = lax.axis_index("s") * NUM_CORES + lax.axis_index("c")
        wbase = wid * B_PER_W
        for c in range(N_CHUNKS):
            sl = pl.ds(wbase + c * CHUNK, CHUNK)
            pltpu.sync_copy(h_i.at[sl], ih)
            pltpu.sync_copy(t_i.at[sl], it)
            pltpu.sync_copy(l_i.at[sl], il)
            pltpu.sync_copy(ha_i.at[sl], iha)
            pltpu.sync_copy(ta_i.at[sl], ita)
            pltpu.sync_copy(la_i.at[sl], ila)
            cps = [
                pltpu.async_copy(ent.at[ih], rh, sem),
                pltpu.async_copy(ent.at[it], rt, sem),
                pltpu.async_copy(rel.at[il], rl, sem),
                pltpu.async_copy(ent.at[iha], rha, sem),
                pltpu.async_copy(ent.at[ita], rta, sem),
                pltpu.async_copy(rel.at[ila], rla, sem),
            ]
            for cp in cps:
                cp.wait()
            pltpu.sync_copy(rh, hv_o.at[sl])
            pltpu.sync_copy(rt, tv_o.at[sl])
            pltpu.sync_copy(rha, hav_o.at[sl])
            pltpu.sync_copy(rta, tav_o.at[sl])
            _dist_chunk(rh, rl, rt, db)
            _dist_chunk(rha, rla, rta, dab)
            pltpu.sync_copy(db, dist_o.at[sl])
            pltpu.sync_copy(dab, dista_o.at[sl])

    return trans_e


_TRANS_E = _make_kernel()


def kernel(h_batch, t_batch, l_batch, h_apos_batch, t_apos_batch,
           l_apos_batch, entity_embedding, relation_embedding):
    i32 = jnp.int32
    return _TRANS_E(
        h_batch.astype(i32), t_batch.astype(i32), l_batch.astype(i32),
        h_apos_batch.astype(i32), t_apos_batch.astype(i32),
        l_apos_batch.astype(i32), entity_embedding, relation_embedding)


# speedup vs baseline: 1.0770x; 1.0770x over previous
import functools

import jax
import jax.numpy as jnp
from jax import lax
from jax.experimental import pallas as pl
from jax.experimental.pallas import tpu as pltpu
from jax.experimental.pallas import tpu_sc as plsc

E_DIM = 64
ROW_PAD = 128
BATCH = 16384
NUM_CORES = 2
NUM_SUBCORES = 16
NUM_WORKERS = NUM_CORES * NUM_SUBCORES
B_PER_W = BATCH // NUM_WORKERS
CHUNK = 128
N_CHUNKS = B_PER_W // CHUNK
LANES = 16


def _vsqrt(x):
    xi = lax.bitcast_convert_type(x, jnp.int32)
    yi = lax.shift_right_logical(xi, 1) + jnp.int32(0x1FBD1DF5)
    y = lax.bitcast_convert_type(yi, jnp.float32)
    for _ in range(3):
        y = 0.5 * (y + x / y)
    return y


def _dist_chunk(h_ref, l_ref, t_ref, out_ref):
    iota = lax.iota(jnp.int32, LANES)

    def group_body(g, _):
        def row_body(i, acc):
            r = g * LANES + i
            p = jnp.zeros((LANES,), jnp.float32)
            for k in range(E_DIM // LANES):
                sl = pl.ds(k * LANES, LANES)
                e = h_ref[r, sl] + l_ref[r, sl] - t_ref[r, sl]
                p = p + e * e
            s = jnp.sum(p)
            return jnp.where(iota == i, s, acc)

        acc = lax.fori_loop(0, LANES, row_body,
                            jnp.zeros((LANES,), jnp.float32))
        out_ref[pl.ds(g * LANES, LANES)] = _vsqrt(acc)
        return 0

    lax.fori_loop(0, CHUNK // LANES, group_body, 0)


def _make_kernel():
    f32 = jnp.float32
    mesh = plsc.VectorSubcoreMesh(core_axis_name="c", subcore_axis_name="s")
    out_type = (
        jax.ShapeDtypeStruct((BATCH,), f32),
        jax.ShapeDtypeStruct((BATCH,), f32),
        jax.ShapeDtypeStruct((BATCH, E_DIM), f32),
        jax.ShapeDtypeStruct((BATCH, E_DIM), f32),
        jax.ShapeDtypeStruct((BATCH, E_DIM), f32),
        jax.ShapeDtypeStruct((BATCH, E_DIM), f32),
    )
    scratch = [
        pltpu.VMEM((CHUNK,), jnp.int32),
        pltpu.VMEM((CHUNK,), jnp.int32),
        pltpu.VMEM((CHUNK,), jnp.int32),
        pltpu.VMEM((CHUNK,), jnp.int32),
        pltpu.VMEM((CHUNK,), jnp.int32),
        pltpu.VMEM((CHUNK,), jnp.int32),
        pltpu.VMEM((CHUNK, ROW_PAD), f32),
        pltpu.VMEM((CHUNK, ROW_PAD), f32),
        pltpu.VMEM((CHUNK, ROW_PAD), f32),
        pltpu.VMEM((CHUNK, ROW_PAD), f32),
        pltpu.VMEM((CHUNK, ROW_PAD), f32),
        pltpu.VMEM((CHUNK, ROW_PAD), f32),
        pltpu.VMEM((CHUNK,), f32),
        pltpu.VMEM((CHUNK,), f32),
        pltpu.SemaphoreType.DMA,
    ]

    @functools.partial(pl.kernel, mesh=mesh, out_type=out_type,
                       scratch_types=scratch,
                       compiler_params=pltpu.CompilerParams(
                           needs_layout_passes=False,
                           use_tc_tiling_on_sc=False))
    def trans_e(h_i, t_i, l_i, ha_i, ta_i, la_i, ent, rel,
                dist_o, dista_o, hv_o, tv_o, hav_o, tav_o,
                ih, it, il, iha, ita, ila,
                rh, rt, rl, rha, rta, rla,
                db, dab, sem):
        wid = lax.axis_index("s") * NUM_CORES + lax.axis_index("c")
        wbase = wid * B_PER_W
        for c in range(N_CHUNKS):
            sl = pl.ds(wbase + c * CHUNK, CHUNK)
            pltpu.sync_copy(h_i.at[sl], ih)
            pltpu.sync_copy(t_i.at[sl], it)
            pltpu.sync_copy(l_i.at[sl], il)
            pltpu.sync_copy(ha_i.at[sl], iha)
            pltpu.sync_copy(ta_i.at[sl], ita)
            pltpu.sync_copy(la_i.at[sl], ila)
            cps = [
                pltpu.async_copy(ent.at[ih], rh, sem),
                pltpu.async_copy(ent.at[it], rt, sem),
                pltpu.async_copy(rel.at[il], rl, sem),
                pltpu.async_copy(ent.at[iha], rha, sem),
                pltpu.async_copy(ent.at[ita], rta, sem),
                pltpu.async_copy(rel.at[ila], rla, sem),
            ]
            for cp in cps:
                cp.wait()
            dcol = pl.ds(0, E_DIM)
            pltpu.sync_copy(rh.at[:, dcol], hv_o.at[sl])
            pltpu.sync_copy(rt.at[:, dcol], tv_o.at[sl])
            pltpu.sync_copy(rha.at[:, dcol], hav_o.at[sl])
            pltpu.sync_copy(rta.at[:, dcol], tav_o.at[sl])
            _dist_chunk(rh, rl, rt, db)
            _dist_chunk(rha, rla, rta, dab)
            pltpu.sync_copy(db, dist_o.at[sl])
            pltpu.sync_copy(dab, dista_o.at[sl])

    return trans_e


_TRANS_E = _make_kernel()


def kernel(h_batch, t_batch, l_batch, h_apos_batch, t_apos_batch,
           l_apos_batch, entity_embedding, relation_embedding):
    i32 = jnp.int32
    pad = ((0, 0), (0, ROW_PAD - E_DIM))
    ent = jnp.pad(entity_embedding, pad)
    rel = jnp.pad(relation_embedding, pad)
    return _TRANS_E(
        h_batch.astype(i32), t_batch.astype(i32), l_batch.astype(i32),
        h_apos_batch.astype(i32), t_apos_batch.astype(i32),
        l_apos_batch.astype(i32), ent, rel)


# speedup vs baseline: 1.5809x; 1.4679x over previous
import functools

import jax
import jax.numpy as jnp
from jax import lax
from jax.experimental import pallas as pl
from jax.experimental.pallas import tpu as pltpu
from jax.experimental.pallas import tpu_sc as plsc
from jax.experimental.layout import Format, Layout, with_layout_constraint

E_DIM = 64
ROW_PAD = 128
BATCH = 16384
NUM_CORES = 2
NUM_SUBCORES = 16
NUM_WORKERS = NUM_CORES * NUM_SUBCORES
B_PER_W = BATCH // NUM_WORKERS
CHUNK = 128
N_CHUNKS = B_PER_W // CHUNK
LANES = 16


def _vsqrt(x):
    xi = lax.bitcast_convert_type(x, jnp.int32)
    yi = lax.shift_right_logical(xi, 1) + jnp.int32(0x1FBD1DF5)
    y = lax.bitcast_convert_type(yi, jnp.float32)
    for _ in range(3):
        y = 0.5 * (y + x / y)
    return y


def _dist_chunk(h_ref, l_ref, t_ref, out_ref):
    iota = lax.iota(jnp.int32, LANES)

    def group_body(g, _):
        def row_body(i, acc):
            r = g * LANES + i
            p = jnp.zeros((LANES,), jnp.float32)
            for k in range(E_DIM // LANES):
                sl = pl.ds(k * LANES, LANES)
                e = h_ref[r, sl] + l_ref[r, sl] - t_ref[r, sl]
                p = p + e * e
            s = jnp.sum(p)
            return jnp.where(iota == i, s, acc)

        acc = lax.fori_loop(0, LANES, row_body,
                            jnp.zeros((LANES,), jnp.float32))
        out_ref[pl.ds(g * LANES, LANES)] = _vsqrt(acc)
        return 0

    lax.fori_loop(0, CHUNK // LANES, group_body, 0)


def _make_kernel():
    f32 = jnp.float32
    mesh = plsc.VectorSubcoreMesh(core_axis_name="c", subcore_axis_name="s")
    out_type = (
        jax.ShapeDtypeStruct((BATCH,), f32),
        jax.ShapeDtypeStruct((BATCH,), f32),
        jax.ShapeDtypeStruct((BATCH, E_DIM), f32),
        jax.ShapeDtypeStruct((BATCH, E_DIM), f32),
        jax.ShapeDtypeStruct((BATCH, E_DIM), f32),
        jax.ShapeDtypeStruct((BATCH, E_DIM), f32),
    )
    scratch = [
        pltpu.VMEM((CHUNK,), jnp.int32),
        pltpu.VMEM((CHUNK,), jnp.int32),
        pltpu.VMEM((CHUNK,), jnp.int32),
        pltpu.VMEM((CHUNK,), jnp.int32),
        pltpu.VMEM((CHUNK,), jnp.int32),
        pltpu.VMEM((CHUNK,), jnp.int32),
        pltpu.VMEM((CHUNK, E_DIM), f32),
        pltpu.VMEM((CHUNK, E_DIM), f32),
        pltpu.VMEM((CHUNK, E_DIM), f32),
        pltpu.VMEM((CHUNK, E_DIM), f32),
        pltpu.VMEM((CHUNK, E_DIM), f32),
        pltpu.VMEM((CHUNK, E_DIM), f32),
        pltpu.VMEM((CHUNK,), f32),
        pltpu.VMEM((CHUNK,), f32),
        pltpu.SemaphoreType.DMA,
    ]

    @functools.partial(pl.kernel, mesh=mesh, out_type=out_type,
                       scratch_types=scratch,
                       compiler_params=pltpu.CompilerParams(
                           needs_layout_passes=False,
                           use_tc_tiling_on_sc=False))
    def trans_e(h_i, t_i, l_i, ha_i, ta_i, la_i, ent, rel,
                dist_o, dista_o, hv_o, tv_o, hav_o, tav_o,
                ih, it, il, iha, ita, ila,
                rh, rt, rl, rha, rta, rla,
                db, dab, sem):
        wid = lax.axis_index("s") * NUM_CORES + lax.axis_index("c")
        wbase = wid * B_PER_W
        for c in range(N_CHUNKS):
            sl = pl.ds(wbase + c * CHUNK, CHUNK)
            pltpu.sync_copy(h_i.at[sl], ih)
            pltpu.sync_copy(t_i.at[sl], it)
            pltpu.sync_copy(l_i.at[sl], il)
            pltpu.sync_copy(ha_i.at[sl], iha)
            pltpu.sync_copy(ta_i.at[sl], ita)
            pltpu.sync_copy(la_i.at[sl], ila)
            cps = [
                pltpu.async_copy(ent.at[ih], rh, sem),
                pltpu.async_copy(ent.at[it], rt, sem),
                pltpu.async_copy(rel.at[il], rl, sem),
                pltpu.async_copy(ent.at[iha], rha, sem),
                pltpu.async_copy(ent.at[ita], rta, sem),
                pltpu.async_copy(rel.at[ila], rla, sem),
            ]
            for cp in cps:
                cp.wait()
            pltpu.sync_copy(rh, hv_o.at[sl])
            pltpu.sync_copy(rt, tv_o.at[sl])
            pltpu.sync_copy(rha, hav_o.at[sl])
            pltpu.sync_copy(rta, tav_o.at[sl])
            _dist_chunk(rh, rl, rt, db)
            _dist_chunk(rha, rla, rta, dab)
            pltpu.sync_copy(db, dist_o.at[sl])
            pltpu.sync_copy(dab, dista_o.at[sl])

    return trans_e


_TRANS_E = _make_kernel()


def kernel(h_batch, t_batch, l_batch, h_apos_batch, t_apos_batch,
           l_apos_batch, entity_embedding, relation_embedding):
    i32 = jnp.int32
    fmt = Layout((0, 1), tiling=((8,),))
    ent = with_layout_constraint(entity_embedding, fmt)
    rel = with_layout_constraint(relation_embedding, fmt)
    return _TRANS_E(
        h_batch.astype(i32), t_batch.astype(i32), l_batch.astype(i32),
        h_apos_batch.astype(i32), t_apos_batch.astype(i32),
        l_apos_batch.astype(i32), ent, rel)
